# baseline (speedup 1.0000x reference)
import jax
import jax.numpy as jnp
from jax.experimental import pallas as pl


def _eye(n):
    r = jax.lax.broadcasted_iota(jnp.int32, (n, n), 0)
    c = jax.lax.broadcasted_iota(jnp.int32, (n, n), 1)
    return (r == c).astype(jnp.float32)


def _t_small(x):
    return jax.lax.dot_general(
        _eye(x.shape[1]), x, (((1,), (1,)), ((), ())),
        preferred_element_type=jnp.float32)


def _stage1_body(loc_ref, conf_ref, db_ref, gtb_ref, gtl_ref,
                 lossc_ref, aux_ref):
    G = gtb_ref.shape[1]
    P, C = conf_ref.shape[1], conf_ref.shape[2]

    db = db_ref[...]
    dx0, dy0 = db[0:1, :], db[1:2, :]
    dx1, dy1 = db[2:3, :], db[3:4, :]
    area_d = (dx1 - dx0) * (dy1 - dy0)

    gtb = gtb_ref[0]
    tx0, ty0 = gtb[:, 0:1], gtb[:, 1:2]
    tx1, ty1 = gtb[:, 2:3], gtb[:, 3:4]
    area_t = (tx1 - tx0) * (ty1 - ty0)

    ltx = jnp.maximum(dx0, tx0)
    lty = jnp.maximum(dy0, ty0)
    rbx = jnp.minimum(dx1, tx1)
    rby = jnp.minimum(dy1, ty1)
    w = jnp.maximum(rbx - ltx, 0.0)
    h = jnp.maximum(rby - lty, 0.0)
    inter = w * h
    iou = inter / (area_d + area_t - inter)

    best_ov = jnp.max(iou, axis=0, keepdims=True)
    gidx = jax.lax.broadcasted_iota(jnp.int32, (G, P), 0)
    best_idx = jnp.min(jnp.where(iou == best_ov, gidx, G),
                       axis=0, keepdims=True)

    bt = jnp.max(iou, axis=1, keepdims=True)
    pidx = jax.lax.broadcasted_iota(jnp.int32, (G, P), 1)
    bp_idx = jnp.min(jnp.where(iou == bt, pidx, P),
                     axis=1, keepdims=True)

    piota = jax.lax.broadcasted_iota(jnp.int32, (1, P), 1)
    gcol = jax.lax.broadcasted_iota(jnp.int32, (G, 1), 0)
    forced = jnp.max(jnp.where(bp_idx == piota, gcol, -1),
                     axis=0, keepdims=True)
    fm = forced >= 0
    best_idx = jnp.where(fm, forced, best_idx)
    best_ov = jnp.where(fm, 2.0, best_ov)
    pos = best_ov >= 0.5
    posf = pos.astype(jnp.float32)

    selg = best_idx == gcol

    def gsel(col):
        return jnp.sum(jnp.where(selg, col, 0.0), axis=0, keepdims=True)

    mx0, my0, mx1, my1 = gsel(tx0), gsel(ty0), gsel(tx1), gsel(ty1)
    labs = gtl_ref[0].astype(jnp.int32)
    lab = jnp.sum(jnp.where(selg, labs, 0), axis=0, keepdims=True)
    conf_label = jnp.where(pos, lab, 0)

    gw, gh = mx1 - mx0, my1 - my0
    gcx, gcy = mx0 + gw * 0.5, my0 + gh * 0.5
    dw, dh = dx1 - dx0, dy1 - dy0
    dcx, dcy = dx0 + dw * 0.5, dy0 + dh * 0.5
    e0 = (gcx - dcx) / (dw + 1e-8)
    e1 = (gcy - dcy) / (dh + 1e-8)
    e2 = jnp.log(gw / (dw + 1e-8) + 1e-8)
    e3 = jnp.log(gh / (dh + 1e-8) + 1e-8)

    loc = _t_small(loc_ref[0])

    def sl1(pred, tgt):
        d = pred - tgt
        ad = jnp.abs(d)
        return jnp.where(ad < 1.0, 0.5 * d * d, ad - 0.5)

    l1 = (sl1(loc[0:1, :], e0) + sl1(loc[1:2, :], e1)
          + sl1(loc[2:3, :], e2) + sl1(loc[3:4, :], e3))
    loc_l = jnp.sum(l1 * posf)

    conf = _t_small(conf_ref[0])
    m = jnp.max(conf, axis=0, keepdims=True)
    lse = m + jnp.log(jnp.sum(jnp.exp(conf - m), axis=0, keepdims=True))
    ccol = jax.lax.broadcasted_iota(jnp.int32, (C, 1), 0)
    picked = jnp.sum(jnp.where(conf_label == ccol, conf, 0.0),
                     axis=0, keepdims=True)
    ce = lse - picked
    ce_pos = jnp.sum(ce * posf)
    npos = jnp.sum(posf)

    lossc_ref[0] = jnp.where(pos, 0.0, ce)
    lane = jax.lax.broadcasted_iota(jnp.int32, (1, 128), 1)
    aux_ref[0] = jnp.where(lane == 0, loc_l,
                           jnp.where(lane == 1, ce_pos,
                                     jnp.where(lane == 2, npos, 0.0)))


def _stage2_body(lossc_ref, aux_ref, out_ref):
    B = lossc_ref.shape[0]
    P = lossc_ref.shape[2]
    v = lossc_ref[...][:, 0, :]
    aux = aux_ref[...][:, 0, :]
    loc_l = jnp.sum(aux[:, 0:1])
    ce_pos = jnp.sum(aux[:, 1:2])
    npos_col = aux[:, 2:3]
    npos_tot = jnp.sum(npos_col)
    k = jnp.clip(3 * npos_col.astype(jnp.int32), 1, P - 1)

    vi = jax.lax.bitcast_convert_type(v, jnp.int32)

    def body(i, carry):
        prefix, need = carry
        bit = 31 - i
        bitv = jnp.left_shift(jnp.int32(1), bit)
        mask_hi = jnp.left_shift(jnp.int32(-1), bit)
        cand = prefix | bitv
        cnt = jnp.sum(((vi & mask_hi) == cand).astype(jnp.int32),
                      axis=1, keepdims=True)
        take = need <= cnt
        prefix = jnp.where(take, cand, prefix)
        need = jnp.where(take, need, need - cnt)
        return prefix, need

    init = (jnp.zeros((B, 1), jnp.int32), k)
    prefix, _ = jax.lax.fori_loop(0, 32, body, init)
    thr_f = jax.lax.bitcast_convert_type(prefix, jnp.float32)
    gt = vi > prefix
    sum_gt = jnp.sum(jnp.where(gt, v, 0.0), axis=1, keepdims=True)
    cnt_gt = jnp.sum(gt.astype(jnp.int32), axis=1, keepdims=True)
    topk = sum_gt + (k - cnt_gt).astype(jnp.float32) * thr_f

    total = loc_l + ce_pos + jnp.sum(topk)
    out_ref[...] = jnp.broadcast_to(total / jnp.maximum(npos_tot, 1.0), (1, 1))


def kernel(loc_preds, conf_preds, default_boxes, gt_boxes, gt_labels):
    B, P, C = conf_preds.shape
    G = gt_boxes.shape[1]

    db_t = default_boxes.T
    gtl = gt_labels.astype(jnp.int32)[..., None]

    loss_c, aux = pl.pallas_call(
        _stage1_body,
        grid=(B,),
        in_specs=[
            pl.BlockSpec((1, P, 4), lambda b: (b, 0, 0)),
            pl.BlockSpec((1, P, C), lambda b: (b, 0, 0)),
            pl.BlockSpec((4, P), lambda b: (0, 0)),
            pl.BlockSpec((1, G, 4), lambda b: (b, 0, 0)),
            pl.BlockSpec((1, G, 1), lambda b: (b, 0, 0)),
        ],
        out_specs=[
            pl.BlockSpec((1, 1, P), lambda b: (b, 0, 0)),
            pl.BlockSpec((1, 1, 128), lambda b: (b, 0, 0)),
        ],
        out_shape=[
            jax.ShapeDtypeStruct((B, 1, P), jnp.float32),
            jax.ShapeDtypeStruct((B, 1, 128), jnp.float32),
        ],
    )(loc_preds, conf_preds, db_t, gt_boxes, gtl)

    out = pl.pallas_call(
        _stage2_body,
        in_specs=[
            pl.BlockSpec((B, 1, P), lambda: (0, 0, 0)),
            pl.BlockSpec((B, 1, 128), lambda: (0, 0, 0)),
        ],
        out_specs=pl.BlockSpec((1, 1), lambda: (0, 0)),
        out_shape=jax.ShapeDtypeStruct((1, 1), jnp.float32),
    )(loss_c, aux)
    return out[0, 0]


# speedup vs baseline: 1.6880x; 1.6880x over previous
import jax
import jax.numpy as jnp
from jax.experimental import pallas as pl


def _match_body(db_ref, gtb_ref, gtl_ref, lab_ref, enc_ref, aux_ref):
    G = gtb_ref.shape[1]
    P = db_ref.shape[1]

    db = db_ref[...]
    dx0, dy0 = db[0:1, :], db[1:2, :]
    dx1, dy1 = db[2:3, :], db[3:4, :]
    area_d = (dx1 - dx0) * (dy1 - dy0)

    gtb = gtb_ref[0]
    tx0, ty0 = gtb[:, 0:1], gtb[:, 1:2]
    tx1, ty1 = gtb[:, 2:3], gtb[:, 3:4]
    area_t = (tx1 - tx0) * (ty1 - ty0)

    ltx = jnp.maximum(dx0, tx0)
    lty = jnp.maximum(dy0, ty0)
    rbx = jnp.minimum(dx1, tx1)
    rby = jnp.minimum(dy1, ty1)
    w = jnp.maximum(rbx - ltx, 0.0)
    h = jnp.maximum(rby - lty, 0.0)
    inter = w * h
    iou = inter / (area_d + area_t - inter)

    best_ov = jnp.max(iou, axis=0, keepdims=True)
    gidx = jax.lax.broadcasted_iota(jnp.int32, (G, P), 0)
    best_idx = jnp.min(jnp.where(iou == best_ov, gidx, G),
                       axis=0, keepdims=True)

    bt = jnp.max(iou, axis=1, keepdims=True)
    pidx = jax.lax.broadcasted_iota(jnp.int32, (G, P), 1)
    bp_idx = jnp.min(jnp.where(iou == bt, pidx, P),
                     axis=1, keepdims=True)

    piota = jax.lax.broadcasted_iota(jnp.int32, (1, P), 1)
    gcol = jax.lax.broadcasted_iota(jnp.int32, (G, 1), 0)
    forced = jnp.max(jnp.where(bp_idx == piota, gcol, -1),
                     axis=0, keepdims=True)
    fm = forced >= 0
    best_idx = jnp.where(fm, forced, best_idx)
    best_ov = jnp.where(fm, 2.0, best_ov)
    pos = best_ov >= 0.5
    posf = pos.astype(jnp.float32)

    selg = best_idx == gcol

    def gsel(col):
        return jnp.sum(jnp.where(selg, col, 0.0), axis=0, keepdims=True)

    mx0, my0, mx1, my1 = gsel(tx0), gsel(ty0), gsel(tx1), gsel(ty1)
    labs = gtl_ref[0].astype(jnp.int32)
    lab = jnp.sum(jnp.where(selg, labs, 0), axis=0, keepdims=True)
    conf_label = jnp.where(pos, lab, 0)

    gw, gh = mx1 - mx0, my1 - my0
    gcx, gcy = mx0 + gw * 0.5, my0 + gh * 0.5
    dw, dh = dx1 - dx0, dy1 - dy0
    dcx, dcy = dx0 + dw * 0.5, dy0 + dh * 0.5
    e0 = (gcx - dcx) / (dw + 1e-8)
    e1 = (gcy - dcy) / (dh + 1e-8)
    e2 = jnp.log(gw / (dw + 1e-8) + 1e-8)
    e3 = jnp.log(gh / (dh + 1e-8) + 1e-8)

    lab_ref[0] = conf_label
    enc_ref[0] = jnp.concatenate([e0, e1, e2, e3], axis=0)
    npos = jnp.sum(posf)
    lane = jax.lax.broadcasted_iota(jnp.int32, (1, 128), 1)
    aux_ref[0] = jnp.where(lane == 0, npos, 0.0)


def _loss_body(loc_ref, conf_ref, lab_ref, enc_ref, lossc_ref, aux_ref):
    C = conf_ref.shape[1]

    conf_label = lab_ref[0]
    pos = conf_label > 0
    posf = pos.astype(jnp.float32)

    loc = loc_ref[0]
    enc = enc_ref[0]
    d = loc - enc
    ad = jnp.abs(d)
    sl1 = jnp.where(ad < 1.0, 0.5 * d * d, ad - 0.5)
    loc_l = jnp.sum(jnp.sum(sl1, axis=0, keepdims=True) * posf)

    conf = conf_ref[0]
    m = jnp.max(conf, axis=0, keepdims=True)
    lse = m + jnp.log(jnp.sum(jnp.exp(conf - m), axis=0, keepdims=True))
    ccol = jax.lax.broadcasted_iota(jnp.int32, (C, 1), 0)
    picked = jnp.sum(jnp.where(conf_label == ccol, conf, 0.0),
                     axis=0, keepdims=True)
    ce = lse - picked
    ce_pos = jnp.sum(ce * posf)

    lossc_ref[0] = jnp.where(pos, 0.0, ce)
    lane = jax.lax.broadcasted_iota(jnp.int32, (1, 128), 1)
    aux_ref[0] = jnp.where(lane == 0, loc_l,
                           jnp.where(lane == 1, ce_pos, 0.0))


def _mine_body(lossc_ref, auxa_ref, auxb_ref, out_ref):
    B = lossc_ref.shape[0]
    P = lossc_ref.shape[2]
    v = lossc_ref[...][:, 0, :]
    auxa = auxa_ref[...][:, 0, :]
    auxb = auxb_ref[...][:, 0, :]
    npos_col = auxa[:, 0:1]
    loc_l = jnp.sum(auxb[:, 0:1])
    ce_pos = jnp.sum(auxb[:, 1:2])
    npos_tot = jnp.sum(npos_col)
    k = jnp.clip(3 * npos_col.astype(jnp.int32), 1, P - 1)

    vi = jax.lax.bitcast_convert_type(v, jnp.int32)

    def body(i, carry):
        prefix, need = carry
        bit = 31 - i
        bitv = jnp.left_shift(jnp.int32(1), bit)
        mask_hi = jnp.left_shift(jnp.int32(-1), bit)
        cand = prefix | bitv
        cnt = jnp.sum(((vi & mask_hi) == cand).astype(jnp.int32),
                      axis=1, keepdims=True)
        take = need <= cnt
        prefix = jnp.where(take, cand, prefix)
        need = jnp.where(take, need, need - cnt)
        return prefix, need

    init = (jnp.zeros((B, 1), jnp.int32), k)
    prefix, _ = jax.lax.fori_loop(0, 32, body, init)
    thr_f = jax.lax.bitcast_convert_type(prefix, jnp.float32)
    gt = vi > prefix
    sum_gt = jnp.sum(jnp.where(gt, v, 0.0), axis=1, keepdims=True)
    cnt_gt = jnp.sum(gt.astype(jnp.int32), axis=1, keepdims=True)
    topk = sum_gt + (k - cnt_gt).astype(jnp.float32) * thr_f

    total = loc_l + ce_pos + jnp.sum(topk)
    out_ref[...] = jnp.broadcast_to(total / jnp.maximum(npos_tot, 1.0), (1, 1))


def kernel(loc_preds, conf_preds, default_boxes, gt_boxes, gt_labels):
    B, P, C = conf_preds.shape
    G = gt_boxes.shape[1]

    conf_t = jnp.transpose(conf_preds, (0, 2, 1))
    loc_t = jnp.transpose(loc_preds, (0, 2, 1))
    db_t = default_boxes.T
    gtl = gt_labels.astype(jnp.int32)[..., None]

    lab, enc, aux_a = pl.pallas_call(
        _match_body,
        grid=(B,),
        in_specs=[
            pl.BlockSpec((4, P), lambda b: (0, 0)),
            pl.BlockSpec((1, G, 4), lambda b: (b, 0, 0)),
            pl.BlockSpec((1, G, 1), lambda b: (b, 0, 0)),
        ],
        out_specs=[
            pl.BlockSpec((1, 1, P), lambda b: (b, 0, 0)),
            pl.BlockSpec((1, 4, P), lambda b: (b, 0, 0)),
            pl.BlockSpec((1, 1, 128), lambda b: (b, 0, 0)),
        ],
        out_shape=[
            jax.ShapeDtypeStruct((B, 1, P), jnp.int32),
            jax.ShapeDtypeStruct((B, 4, P), jnp.float32),
            jax.ShapeDtypeStruct((B, 1, 128), jnp.float32),
        ],
    )(db_t, gt_boxes, gtl)

    loss_c, aux_b = pl.pallas_call(
        _loss_body,
        grid=(B,),
        in_specs=[
            pl.BlockSpec((1, 4, P), lambda b: (b, 0, 0)),
            pl.BlockSpec((1, C, P), lambda b: (b, 0, 0)),
            pl.BlockSpec((1, 1, P), lambda b: (b, 0, 0)),
            pl.BlockSpec((1, 4, P), lambda b: (b, 0, 0)),
        ],
        out_specs=[
            pl.BlockSpec((1, 1, P), lambda b: (b, 0, 0)),
            pl.BlockSpec((1, 1, 128), lambda b: (b, 0, 0)),
        ],
        out_shape=[
            jax.ShapeDtypeStruct((B, 1, P), jnp.float32),
            jax.ShapeDtypeStruct((B, 1, 128), jnp.float32),
        ],
    )(loc_t, conf_t, lab, enc)

    out = pl.pallas_call(
        _mine_body,
        in_specs=[
            pl.BlockSpec((B, 1, P), lambda: (0, 0, 0)),
            pl.BlockSpec((B, 1, 128), lambda: (0, 0, 0)),
            pl.BlockSpec((B, 1, 128), lambda: (0, 0, 0)),
        ],
        out_specs=pl.BlockSpec((1, 1), lambda: (0, 0)),
        out_shape=jax.ShapeDtypeStruct((1, 1), jnp.float32),
    )(loss_c, aux_a, aux_b)
    return out[0, 0]


# speedup vs baseline: 1.7020x; 1.0083x over previous
import jax
import jax.numpy as jnp
from jax.experimental import pallas as pl
from jax.experimental.pallas import tpu as pltpu


def _match_body(db_ref, gtb_ref, gtl_ref, lab_ref, enc_ref, aux_ref):
    G = gtb_ref.shape[1]
    P = db_ref.shape[1]

    db = db_ref[...]
    dx0, dy0 = db[0:1, :], db[1:2, :]
    dx1, dy1 = db[2:3, :], db[3:4, :]
    area_d = (dx1 - dx0) * (dy1 - dy0)

    gtb = gtb_ref[0]
    tx0, ty0 = gtb[:, 0:1], gtb[:, 1:2]
    tx1, ty1 = gtb[:, 2:3], gtb[:, 3:4]
    area_t = (tx1 - tx0) * (ty1 - ty0)

    ltx = jnp.maximum(dx0, tx0)
    lty = jnp.maximum(dy0, ty0)
    rbx = jnp.minimum(dx1, tx1)
    rby = jnp.minimum(dy1, ty1)
    w = jnp.maximum(rbx - ltx, 0.0)
    h = jnp.maximum(rby - lty, 0.0)
    inter = w * h
    iou = inter / (area_d + area_t - inter)

    best_ov = jnp.max(iou, axis=0, keepdims=True)
    gidx = jax.lax.broadcasted_iota(jnp.int32, (G, P), 0)
    best_idx = jnp.min(jnp.where(iou == best_ov, gidx, G),
                       axis=0, keepdims=True)

    bt = jnp.max(iou, axis=1, keepdims=True)
    pidx = jax.lax.broadcasted_iota(jnp.int32, (G, P), 1)
    bp_idx = jnp.min(jnp.where(iou == bt, pidx, P),
                     axis=1, keepdims=True)

    piota = jax.lax.broadcasted_iota(jnp.int32, (1, P), 1)
    gcol = jax.lax.broadcasted_iota(jnp.int32, (G, 1), 0)
    forced = jnp.max(jnp.where(bp_idx == piota, gcol, -1),
                     axis=0, keepdims=True)
    fm = forced >= 0
    best_idx = jnp.where(fm, forced, best_idx)
    best_ov = jnp.where(fm, 2.0, best_ov)
    pos = best_ov >= 0.5
    posf = pos.astype(jnp.float32)

    selg = best_idx == gcol

    def gsel(col):
        return jnp.sum(jnp.where(selg, col, 0.0), axis=0, keepdims=True)

    mx0, my0, mx1, my1 = gsel(tx0), gsel(ty0), gsel(tx1), gsel(ty1)
    labs = gtl_ref[0].astype(jnp.int32)
    lab = jnp.sum(jnp.where(selg, labs, 0), axis=0, keepdims=True)
    conf_label = jnp.where(pos, lab, 0)

    gw, gh = mx1 - mx0, my1 - my0
    gcx, gcy = mx0 + gw * 0.5, my0 + gh * 0.5
    dw, dh = dx1 - dx0, dy1 - dy0
    dcx, dcy = dx0 + dw * 0.5, dy0 + dh * 0.5
    e0 = (gcx - dcx) / (dw + 1e-8)
    e1 = (gcy - dcy) / (dh + 1e-8)
    e2 = jnp.log(gw / (dw + 1e-8) + 1e-8)
    e3 = jnp.log(gh / (dh + 1e-8) + 1e-8)

    lab_ref[0] = conf_label
    enc_ref[0] = jnp.concatenate([e0, e1, e2, e3], axis=0)
    npos = jnp.sum(posf)
    lane = jax.lax.broadcasted_iota(jnp.int32, (1, 128), 1)
    aux_ref[0] = jnp.where(lane == 0, npos, 0.0)


def _loss_mine_body(loc_ref, conf_ref, lab_ref, enc_ref, auxa_ref, out_ref,
                    lossc_scr, aux_scr):
    b = pl.program_id(0)
    B = lossc_scr.shape[0]
    C = conf_ref.shape[1]
    P = conf_ref.shape[2]
    lane = jax.lax.broadcasted_iota(jnp.int32, (1, 128), 1)

    @pl.when(b < B)
    def _per_batch():
        conf_label = lab_ref[0]
        pos = conf_label > 0
        posf = pos.astype(jnp.float32)

        loc = loc_ref[0]
        enc = enc_ref[0]
        d = loc - enc
        ad = jnp.abs(d)
        sl1 = jnp.where(ad < 1.0, 0.5 * d * d, ad - 0.5)
        loc_l = jnp.sum(jnp.sum(sl1, axis=0, keepdims=True) * posf)

        conf = conf_ref[0]
        m = jnp.max(conf, axis=0, keepdims=True)
        lse = m + jnp.log(jnp.sum(jnp.exp(conf - m), axis=0, keepdims=True))
        ccol = jax.lax.broadcasted_iota(jnp.int32, (C, 1), 0)
        picked = jnp.sum(jnp.where(conf_label == ccol, conf, 0.0),
                         axis=0, keepdims=True)
        ce = lse - picked
        ce_pos = jnp.sum(ce * posf)

        lossc_scr[pl.ds(b, 1)] = jnp.where(pos, 0.0, ce)[None]
        aux_scr[pl.ds(b, 1)] = jnp.where(lane == 0, loc_l,
                                         jnp.where(lane == 1, ce_pos, 0.0))[None]

    @pl.when(b == B)
    def _mine():
        v = lossc_scr[...][:, 0, :]
        auxa = auxa_ref[...][:, 0, :]
        auxb = aux_scr[...][:, 0, :]
        npos_col = auxa[:, 0:1]
        loc_l = jnp.sum(auxb[:, 0:1])
        ce_pos = jnp.sum(auxb[:, 1:2])
        npos_tot = jnp.sum(npos_col)
        k = jnp.clip(3 * npos_col.astype(jnp.int32), 1, P - 1)

        vi = jax.lax.bitcast_convert_type(v, jnp.int32)

        def body(i, carry):
            prefix, need = carry
            bit = 31 - i
            bitv = jnp.left_shift(jnp.int32(1), bit)
            mask_hi = jnp.left_shift(jnp.int32(-1), bit)
            cand = prefix | bitv
            cnt = jnp.sum(((vi & mask_hi) == cand).astype(jnp.int32),
                          axis=1, keepdims=True)
            take = need <= cnt
            prefix = jnp.where(take, cand, prefix)
            need = jnp.where(take, need, need - cnt)
            return prefix, need

        init = (jnp.zeros((B, 1), jnp.int32), k)
        prefix, _ = jax.lax.fori_loop(0, 32, body, init)
        thr_f = jax.lax.bitcast_convert_type(prefix, jnp.float32)
        gt = vi > prefix
        sum_gt = jnp.sum(jnp.where(gt, v, 0.0), axis=1, keepdims=True)
        cnt_gt = jnp.sum(gt.astype(jnp.int32), axis=1, keepdims=True)
        topk = sum_gt + (k - cnt_gt).astype(jnp.float32) * thr_f

        total = loc_l + ce_pos + jnp.sum(topk)
        out_ref[...] = jnp.broadcast_to(
            total / jnp.maximum(npos_tot, 1.0), (1, 1))


def kernel(loc_preds, conf_preds, default_boxes, gt_boxes, gt_labels):
    B, P, C = conf_preds.shape
    G = gt_boxes.shape[1]

    conf_t = jnp.transpose(conf_preds, (0, 2, 1))
    loc_t = jnp.transpose(loc_preds, (0, 2, 1))
    db_t = default_boxes.T
    gtl = gt_labels.astype(jnp.int32)[..., None]

    lab, enc, aux_a = pl.pallas_call(
        _match_body,
        grid=(B,),
        in_specs=[
            pl.BlockSpec((4, P), lambda b: (0, 0)),
            pl.BlockSpec((1, G, 4), lambda b: (b, 0, 0)),
            pl.BlockSpec((1, G, 1), lambda b: (b, 0, 0)),
        ],
        out_specs=[
            pl.BlockSpec((1, 1, P), lambda b: (b, 0, 0)),
            pl.BlockSpec((1, 4, P), lambda b: (b, 0, 0)),
            pl.BlockSpec((1, 1, 128), lambda b: (b, 0, 0)),
        ],
        out_shape=[
            jax.ShapeDtypeStruct((B, 1, P), jnp.int32),
            jax.ShapeDtypeStruct((B, 4, P), jnp.float32),
            jax.ShapeDtypeStruct((B, 1, 128), jnp.float32),
        ],
    )(db_t, gt_boxes, gtl)

    bidx = lambda b: (jnp.minimum(b, B - 1), 0, 0)
    out = pl.pallas_call(
        _loss_mine_body,
        grid=(B + 1,),
        in_specs=[
            pl.BlockSpec((1, 4, P), bidx),
            pl.BlockSpec((1, C, P), bidx),
            pl.BlockSpec((1, 1, P), bidx),
            pl.BlockSpec((1, 4, P), bidx),
            pl.BlockSpec((B, 1, 128), lambda b: (0, 0, 0)),
        ],
        out_specs=pl.BlockSpec((1, 1), lambda b: (0, 0)),
        out_shape=jax.ShapeDtypeStruct((1, 1), jnp.float32),
        scratch_shapes=[
            pltpu.VMEM((B, 1, P), jnp.float32),
            pltpu.VMEM((B, 1, 128), jnp.float32),
        ],
    )(loc_t, conf_t, lab, enc, aux_a)
    return out[0, 0]
